# baseline (speedup 1.0000x reference)
import jax, jax.numpy as jnp
from jax.experimental import pallas as pl

N = 10000
H = 8
D = 16
CLAMP = 5.0


def _sqrt_relu(v):
    safe = jnp.where(v > 0, v, 1.0)
    return jnp.where(v > 0, jnp.sqrt(safe), 0.0)


def kernel(x, edge_attr, WQ, bQ, WK, bK, WE, bE, WV, bV, Aw, VeRow, edge_index):
    Q_h = (x @ WQ.T + bQ).reshape(-1, H, D)
    K_h = (x @ WK.T + bK).reshape(-1, H, D)
    V_h = (x @ WV.T + bV).reshape(-1, H, D)
    Eproj = edge_attr @ WE.T + bE
    src_i = edge_index[0]
    dst_i = edge_index[1]
    src = jnp.take(K_h, src_i, axis=0)
    dest = jnp.take(Q_h, dst_i, axis=0)
    score1 = src + dest
    Ex = Eproj.reshape(-1, H, 2 * D)
    Ex1 = Ex[:, :, :D]
    Ex2 = Ex[:, :, D:]
    score2 = Ex1 * Ex2
    score = score1 + _sqrt_relu(score2) - _sqrt_relu(-score2)
    e_t = score
    oE = score.reshape(score.shape[0], -1)
    s = jnp.einsum('ehd,dhc->ehc', score, Aw)
    s = jnp.clip(s, -CLAMP, CLAMP)
    smax = jax.ops.segment_max(s, dst_i, num_segments=N)
    out = jnp.exp(s - jnp.take(smax, dst_i, axis=0))
    denom = jax.ops.segment_sum(out, dst_i, num_segments=N)
    attn = out / (jnp.take(denom, dst_i, axis=0) + 1e-16)
    msg = jnp.take(V_h, src_i, axis=0) * attn
    oV = jax.ops.segment_sum(msg, dst_i, num_segments=N)
    rowV = jax.ops.segment_sum(e_t * attn, dst_i, num_segments=N)
    rowV = jnp.einsum('nhd,dhc->nhc', rowV, VeRow)
    oV = (oV + rowV).reshape(N, -1)
    return (oV, oE)


# speedup vs baseline: 25.2766x; 25.2766x over previous
import functools

import jax
import jax.numpy as jnp
from jax import lax
from jax.experimental import pallas as pl
from jax.experimental.pallas import tpu as pltpu
from jax.experimental.pallas import tpu_sc as plsc

H = 8
D = 16
HD = H * D
CLAMP = 5.0

NC = 2
NS = 16
NW = NC * NS


def _tc1_body(x_ref, wqt, wkt, wvt, bq, bk, bv, m, awm128, kf_o, qf_o, vpk_o,
              qv2_o, tka_o, tqa_o):
    xb = x_ref[...]
    kf = jnp.dot(xb, wkt[...], preferred_element_type=jnp.float32) + bk[...]
    qf = jnp.dot(xb, wqt[...], preferred_element_type=jnp.float32) + bq[...]
    vf = jnp.dot(xb, wvt[...], preferred_element_type=jnp.float32) + bv[...]
    kf_o[...] = kf
    qf_o[...] = qf
    vpk_o[...] = vf + jnp.dot(kf, m[...], preferred_element_type=jnp.float32)
    qv2_o[...] = jnp.dot(qf, m[...], preferred_element_type=jnp.float32)
    tka_o[...] = jnp.dot(kf, awm128[...], preferred_element_type=jnp.float32)
    tqa_o[...] = jnp.dot(qf, awm128[...], preferred_element_type=jnp.float32)


def _tc1(x, wqt, wkt, wvt, bq, bk, bv, m, awm128, nb):
    n = x.shape[0]
    blk = n // nb
    full = lambda s: pl.BlockSpec(s, lambda i: (0,) * len(s))
    row = pl.BlockSpec((blk, HD), lambda i: (i, 0))
    f = jnp.float32
    return pl.pallas_call(
        _tc1_body,
        grid=(nb,),
        in_specs=[row, full((HD, HD)), full((HD, HD)), full((HD, HD)),
                  full((1, HD)), full((1, HD)), full((1, HD)),
                  full((HD, HD)), full((HD, HD))],
        out_specs=[row, row, row, row, row, row],
        out_shape=[jax.ShapeDtypeStruct((n, HD), f)] * 6,
    )(x, wqt, wkt, wvt, bq, bk, bv, m, awm128)


def _tc2_body(ea_ref, we1t, we2t, be1, be2, m, awm, s2_o, s2v_o, eaa_o):
    eb = ea_ref[...]
    ex1 = jnp.dot(eb, we1t[...], preferred_element_type=jnp.float32) + be1[...]
    ex2 = jnp.dot(eb, we2t[...], preferred_element_type=jnp.float32) + be2[...]
    p2 = ex1 * ex2
    s2 = jnp.sign(p2) * jnp.sqrt(jnp.abs(p2))
    s2_o[...] = s2
    s2v_o[...] = jnp.dot(s2, m[...], preferred_element_type=jnp.float32)
    eaa_o[...] = jnp.dot(s2, awm[...], preferred_element_type=jnp.float32)


def _tc2(eattr, we1t, we2t, be1, be2, m, awm, nb):
    e = eattr.shape[0]
    blk = e // nb
    full = lambda s: pl.BlockSpec(s, lambda i: (0,) * len(s))
    row = pl.BlockSpec((blk, HD), lambda i: (i, 0))
    row16 = pl.BlockSpec((blk, 16), lambda i: (i, 0))
    f = jnp.float32
    return pl.pallas_call(
        _tc2_body,
        grid=(nb,),
        in_specs=[row, full((HD, HD)), full((HD, HD)), full((1, HD)),
                  full((1, HD)), full((HD, HD)), full((HD, 16))],
        out_specs=[row, row, row16],
        out_shape=[jax.ShapeDtypeStruct((e, HD), f),
                   jax.ShapeDtypeStruct((e, HD), f),
                   jax.ShapeDtypeStruct((e, 16), f)],
    )(eattr, we1t, we2t, be1, be2, m, awm)


def _sca(src_i, dst_i, tka, tqa, ea16):
    e = src_i.shape[0]
    n = tka.shape[0]
    ew = e // NW
    ca = 40
    nchunks = ew // ca
    ncopy = 10
    rows_per_sub = n // ncopy
    f = jnp.float32
    mesh = plsc.VectorSubcoreMesh(core_axis_name="c", subcore_axis_name="s")

    @functools.partial(
        pl.kernel, mesh=mesh,
        out_type=[jax.ShapeDtypeStruct((e, 16), f),
                  jax.ShapeDtypeStruct((NC, n, HD), f)],
        scratch_types=[
            pltpu.VMEM((ca,), jnp.int32),
            pltpu.VMEM((ca,), jnp.int32),
            pltpu.VMEM((ca, HD), f),
            pltpu.VMEM((ca, HD), f),
            pltpu.VMEM((ca, HD), f),
            pltpu.VMEM((ca, 16), f),
            pltpu.VMEM((ca, 16), f),
            pltpu.VMEM_SHARED((n, HD), f),
            pltpu.SemaphoreType.DMA,
        ],
    )
    def k(src_h, dst_h, tka_h, tqa_h, ea_h, p_o, den_o,
          idx_s, idx_d, tka_b, tqa_b, p_b, p16_b, ea_b, den_acc, sem):
        c = lax.axis_index("c")
        s = lax.axis_index("s")
        wid = s * NC + c
        mask = jnp.where(lax.iota(jnp.int32, 16) < 8, 1.0, 0.0).astype(f)

        def zrow(j, carry):
            for t in range(H):
                p_b[j, pl.ds(t * 16, 16)] = jnp.zeros((16,), f)
            return carry
        lax.fori_loop(0, ca, zrow, 0)

        @pl.when(s < ncopy)
        def _init():
            for t in range(rows_per_sub // ca):
                pltpu.sync_copy(
                    p_b, den_acc.at[pl.ds(pl.multiple_of(s * rows_per_sub + t * ca, 8), ca)])
        plsc.subcore_barrier()

        def chunk(kk, carry):
            base = pl.multiple_of(wid * ew + kk * ca, 8)
            pltpu.sync_copy(src_h.at[pl.ds(base, ca)], idx_s)
            pltpu.sync_copy(dst_h.at[pl.ds(base, ca)], idx_d)
            pltpu.async_copy(tka_h.at[idx_s], tka_b, sem).wait()
            pltpu.async_copy(tqa_h.at[idx_d], tqa_b, sem).wait()
            pltpu.sync_copy(ea_h.at[pl.ds(base, ca)], ea_b)

            def body(j, cc):
                v = (tka_b[j, pl.ds(0, 16)] + tqa_b[j, pl.ds(0, 16)]
                     + ea_b[j])
                v = jnp.minimum(jnp.maximum(v, -CLAMP), CLAMP)
                pv = jnp.exp(v - CLAMP) * mask
                p_b[j, pl.ds(0, 16)] = pv
                p16_b[j] = pv
                return cc
            lax.fori_loop(0, ca, body, 0)
            pltpu.sync_copy(p16_b, p_o.at[pl.ds(base, ca)])
            pltpu.sync_copy(p_b, den_acc.at[idx_d], add=True)
            return carry
        lax.fori_loop(0, nchunks, chunk, 0)
        plsc.subcore_barrier()

        @pl.when(s < ncopy)
        def _out():
            pltpu.sync_copy(
                den_acc.at[pl.ds(pl.multiple_of(s * rows_per_sub, 8), rows_per_sub)],
                den_o.at[c, pl.ds(pl.multiple_of(s * rows_per_sub, 8), rows_per_sub)])

    return k(src_i, dst_i, tka, tqa, ea16)


def _scb(src_i, dst_i, kf, qf, vpk, s2, s2v, p):
    e = src_i.shape[0]
    n = kf.shape[0]
    ew = e // NW
    cb = 40
    nchunks = ew // cb
    ncopy = 10
    rows_per_sub = n // ncopy
    f = jnp.float32
    mesh = plsc.VectorSubcoreMesh(core_axis_name="c", subcore_axis_name="s")

    @functools.partial(
        pl.kernel, mesh=mesh,
        out_type=[jax.ShapeDtypeStruct((e, HD), f),
                  jax.ShapeDtypeStruct((NC, n, HD), f)],
        scratch_types=[
            pltpu.VMEM((cb,), jnp.int32),
            pltpu.VMEM((cb,), jnp.int32),
            pltpu.VMEM((cb, HD), f),
            pltpu.VMEM((cb, HD), f),
            pltpu.VMEM((cb, HD), f),
            pltpu.VMEM((cb, HD), f),
            pltpu.VMEM((cb, HD), f),
            pltpu.VMEM((cb, 16), f),
            pltpu.VMEM_SHARED((n, HD), f),
            pltpu.SemaphoreType.DMA,
        ],
    )
    def k(src_h, dst_h, kf_h, qf_h, vpk_h, s2_h, s2v_h, p_h,
          oe_o, acc_o,
          idx_s, idx_d, kf_b, qf_b, vpk_b, s2_b, s2v_b, p_b,
          acc, sem):
        c = lax.axis_index("c")
        s = lax.axis_index("s")
        wid = s * NC + c

        @pl.when(s < ncopy)
        def _init():
            def zrow(j, carry):
                for t in range(H):
                    vpk_b[j, pl.ds(t * 16, 16)] = jnp.zeros((16,), f)
                return carry
            lax.fori_loop(0, cb, zrow, 0)
            for t in range(rows_per_sub // cb):
                pltpu.sync_copy(
                    vpk_b, acc.at[pl.ds(pl.multiple_of(s * rows_per_sub + t * cb, 8), cb)])
        plsc.subcore_barrier()

        def chunk(kk, carry):
            base = pl.multiple_of(wid * ew + kk * cb, 8)
            pltpu.sync_copy(src_h.at[pl.ds(base, cb)], idx_s)
            pltpu.sync_copy(dst_h.at[pl.ds(base, cb)], idx_d)
            pltpu.async_copy(kf_h.at[idx_s], kf_b, sem).wait()
            pltpu.async_copy(qf_h.at[idx_d], qf_b, sem).wait()
            pltpu.async_copy(vpk_h.at[idx_s], vpk_b, sem).wait()
            pltpu.sync_copy(s2_h.at[pl.ds(base, cb)], s2_b)
            pltpu.sync_copy(s2v_h.at[pl.ds(base, cb)], s2v_b)
            pltpu.sync_copy(p_h.at[pl.ds(base, cb)], p_b)

            def ebody(j, cc):
                pv = p_b[j]
                for h in range(H):
                    sl = pl.ds(h * D, 16)
                    kf_b[j, sl] = kf_b[j, sl] + qf_b[j, sl] + s2_b[j, sl]
                    w = jnp.full((16,), pv[h], f)
                    vpk_b[j, sl] = w * (vpk_b[j, sl] + s2v_b[j, sl])
                return cc
            lax.fori_loop(0, cb, ebody, 0)

            pltpu.sync_copy(kf_b, oe_o.at[pl.ds(base, cb)])
            pltpu.sync_copy(vpk_b, acc.at[idx_d], add=True)
            return carry
        lax.fori_loop(0, nchunks, chunk, 0)
        plsc.subcore_barrier()

        @pl.when(s < ncopy)
        def _out():
            pltpu.sync_copy(
                acc.at[pl.ds(pl.multiple_of(s * rows_per_sub, 8), rows_per_sub)],
                acc_o.at[c, pl.ds(pl.multiple_of(s * rows_per_sub, 8), rows_per_sub)])

    return k(src_i, dst_i, kf, qf, vpk, s2, s2v, p)


def _tc3_body(a0, a1, d0, d1, qv2, r, out):
    den16 = d0[:, :16] + d1[:, :16]
    d128 = jnp.dot(den16, r[...], preferred_element_type=jnp.float32)
    inv = 1.0 / (d128 + 1e-16)
    out[...] = (a0[...] + a1[...]) * inv + (d128 * inv) * qv2[...]


def _tc3(acc0, acc1, den0, den1, qv2, r, nb):
    n = acc0.shape[0]
    blk = n // nb
    full = lambda s: pl.BlockSpec(s, lambda i: (0,) * len(s))
    row = pl.BlockSpec((blk, HD), lambda i: (i, 0))
    return pl.pallas_call(
        _tc3_body,
        grid=(nb,),
        in_specs=[row, row, row, row, row, full((16, HD))],
        out_specs=row,
        out_shape=jax.ShapeDtypeStruct((n, HD), jnp.float32),
    )(acc0, acc1, den0, den1, qv2, r)


def kernel(x, edge_attr, WQ, bQ, WK, bK, WE, bE, WV, bV, Aw, VeRow, edge_index):
    n = x.shape[0]
    e = edge_attr.shape[0]
    f = jnp.float32

    wqt, wkt, wvt = WQ.T, WK.T, WV.T
    we_r = WE.reshape(H, 2, D, HD)
    we1t = we_r[:, 0].reshape(HD, HD).T
    we2t = we_r[:, 1].reshape(HD, HD).T
    be_r = bE.reshape(H, 2, D)
    be1 = be_r[:, 0].reshape(1, HD)
    be2 = be_r[:, 1].reshape(1, HD)
    eye8 = jnp.eye(H, dtype=f)
    m = jnp.einsum('dhc,hg->hdgc', VeRow, eye8).reshape(HD, HD)
    awm = jnp.einsum('dh,hg->hdg', Aw[:, :, 0], eye8).reshape(HD, H)
    awm = jnp.concatenate([awm, jnp.zeros((HD, H), f)], axis=1)
    awm128 = jnp.concatenate([awm, jnp.zeros((HD, HD - 16), f)], axis=1)
    r = jnp.concatenate(
        [jnp.kron(eye8, jnp.ones((1, D), f)), jnp.zeros((H, HD), f)], axis=0)
    bq = bQ.reshape(1, HD)
    bk = bK.reshape(1, HD)
    bv = bV.reshape(1, HD)
    src_i = edge_index[0]
    dst_i = edge_index[1]

    kf, qf, vpk, qv2, tka, tqa = _tc1(x, wqt, wkt, wvt, bq, bk, bv, m,
                                      awm128, 5)
    s2, s2v, ea16 = _tc2(edge_attr, we1t, we2t, be1, be2, m, awm, 160)
    p, denp = _sca(src_i, dst_i, tka, tqa, ea16)
    oe, accp = _scb(src_i, dst_i, kf, qf, vpk, s2, s2v, p)
    ov = _tc3(accp[0], accp[1], denp[0], denp[1], qv2, r, 5)
    return (ov, oe)


# speedup vs baseline: 38.3274x; 1.5163x over previous
import functools

import jax
import jax.numpy as jnp
from jax import lax
from jax.experimental import pallas as pl
from jax.experimental.pallas import tpu as pltpu
from jax.experimental.pallas import tpu_sc as plsc

H = 8
D = 16
HD = H * D
CLAMP = 5.0

NC = 2
NS = 16
NW = NC * NS


def _tc1_body(x_ref, wqt, wkt, wvt, bq, bk, bv, m, awm128, kf_o, qf_o, vpk_o,
              qv2_o, tka_o, tqa_o):
    xb = x_ref[...]
    kf = jnp.dot(xb, wkt[...], preferred_element_type=jnp.float32) + bk[...]
    qf = jnp.dot(xb, wqt[...], preferred_element_type=jnp.float32) + bq[...]
    vf = jnp.dot(xb, wvt[...], preferred_element_type=jnp.float32) + bv[...]
    kf_o[...] = kf
    qf_o[...] = qf
    vpk_o[...] = vf + jnp.dot(kf, m[...], preferred_element_type=jnp.float32)
    qv2_o[...] = jnp.dot(qf, m[...], preferred_element_type=jnp.float32)
    tka_o[...] = jnp.dot(kf, awm128[...], preferred_element_type=jnp.float32)
    tqa_o[...] = jnp.dot(qf, awm128[...], preferred_element_type=jnp.float32)


def _tc1(x, wqt, wkt, wvt, bq, bk, bv, m, awm128, nb):
    n = x.shape[0]
    blk = n // nb
    full = lambda s: pl.BlockSpec(s, lambda i: (0,) * len(s))
    row = pl.BlockSpec((blk, HD), lambda i: (i, 0))
    f = jnp.float32
    return pl.pallas_call(
        _tc1_body,
        grid=(nb,),
        in_specs=[row, full((HD, HD)), full((HD, HD)), full((HD, HD)),
                  full((1, HD)), full((1, HD)), full((1, HD)),
                  full((HD, HD)), full((HD, HD))],
        out_specs=[row, row, row, row, row, row],
        out_shape=[jax.ShapeDtypeStruct((n, HD), f)] * 6,
    )(x, wqt, wkt, wvt, bq, bk, bv, m, awm128)


def _tc2_body(ea_ref, we1t, we2t, be1, be2, m, awm, s2_o, s2v_o, eaa_o):
    eb = ea_ref[...]
    ex1 = jnp.dot(eb, we1t[...], preferred_element_type=jnp.float32) + be1[...]
    ex2 = jnp.dot(eb, we2t[...], preferred_element_type=jnp.float32) + be2[...]
    p2 = ex1 * ex2
    s2 = jnp.sign(p2) * jnp.sqrt(jnp.abs(p2))
    s2_o[...] = s2
    s2v_o[...] = jnp.dot(s2, m[...], preferred_element_type=jnp.float32)
    eaa_o[...] = jnp.dot(s2, awm[...], preferred_element_type=jnp.float32)


def _tc2(eattr, we1t, we2t, be1, be2, m, awm, nb):
    e = eattr.shape[0]
    blk = e // nb
    full = lambda s: pl.BlockSpec(s, lambda i: (0,) * len(s))
    row = pl.BlockSpec((blk, HD), lambda i: (i, 0))
    row16 = pl.BlockSpec((blk, 16), lambda i: (i, 0))
    f = jnp.float32
    return pl.pallas_call(
        _tc2_body,
        grid=(nb,),
        in_specs=[row, full((HD, HD)), full((HD, HD)), full((1, HD)),
                  full((1, HD)), full((HD, HD)), full((HD, 16))],
        out_specs=[row, row, row16],
        out_shape=[jax.ShapeDtypeStruct((e, HD), f),
                   jax.ShapeDtypeStruct((e, HD), f),
                   jax.ShapeDtypeStruct((e, 16), f)],
    )(eattr, we1t, we2t, be1, be2, m, awm)


def _sca(src_i, dst_i, tka, tqa, ea16):
    e = src_i.shape[0]
    n = tka.shape[0]
    ew = e // NW
    ca = 40
    nchunks = ew // ca
    ncopy = 10
    rows_per_sub = n // ncopy
    f = jnp.float32
    mesh = plsc.VectorSubcoreMesh(core_axis_name="c", subcore_axis_name="s")

    @functools.partial(
        pl.kernel, mesh=mesh,
        out_type=[jax.ShapeDtypeStruct((e, 16), f),
                  jax.ShapeDtypeStruct((NC, n, HD), f)],
        scratch_types=[
            pltpu.VMEM((ca,), jnp.int32),
            pltpu.VMEM((ca,), jnp.int32),
            pltpu.VMEM((ca, HD), f),
            pltpu.VMEM((ca, HD), f),
            pltpu.VMEM((ca, HD), f),
            pltpu.VMEM((ca, 16), f),
            pltpu.VMEM((ca, 16), f),
            pltpu.VMEM_SHARED((n, HD), f),
            pltpu.SemaphoreType.DMA,
        ],
    )
    def k(src_h, dst_h, tka_h, tqa_h, ea_h, p_o, den_o,
          idx_s, idx_d, tka_b, tqa_b, p_b, p16_b, ea_b, den_acc, sem):
        c = lax.axis_index("c")
        s = lax.axis_index("s")
        wid = s * NC + c
        mask = jnp.where(lax.iota(jnp.int32, 16) < 8, 1.0, 0.0).astype(f)

        def zrow(j, carry):
            for t in range(H):
                p_b[j, pl.ds(t * 16, 16)] = jnp.zeros((16,), f)
            return carry
        lax.fori_loop(0, ca, zrow, 0)

        @pl.when(s < ncopy)
        def _init():
            for t in range(rows_per_sub // ca):
                pltpu.sync_copy(
                    p_b, den_acc.at[pl.ds(pl.multiple_of(s * rows_per_sub + t * ca, 8), ca)])
        plsc.subcore_barrier()

        def chunk(kk, carry):
            base = pl.multiple_of(wid * ew + kk * ca, 8)
            pltpu.sync_copy(src_h.at[pl.ds(base, ca)], idx_s)
            pltpu.sync_copy(dst_h.at[pl.ds(base, ca)], idx_d)
            d1 = pltpu.async_copy(tka_h.at[idx_s], tka_b, sem)
            d2 = pltpu.async_copy(tqa_h.at[idx_d], tqa_b, sem)
            d3 = pltpu.async_copy(ea_h.at[pl.ds(base, ca)], ea_b, sem)
            d1.wait()
            d2.wait()
            d3.wait()

            def body(j, cc):
                v = (tka_b[j, pl.ds(0, 16)] + tqa_b[j, pl.ds(0, 16)]
                     + ea_b[j])
                v = jnp.minimum(jnp.maximum(v, -CLAMP), CLAMP)
                pv = jnp.exp(v - CLAMP) * mask
                p_b[j, pl.ds(0, 16)] = pv
                p16_b[j] = pv
                return cc
            lax.fori_loop(0, ca, body, 0)
            pltpu.sync_copy(p16_b, p_o.at[pl.ds(base, ca)])
            pltpu.sync_copy(p_b, den_acc.at[idx_d], add=True)
            return carry
        lax.fori_loop(0, nchunks, chunk, 0)
        plsc.subcore_barrier()

        @pl.when(s < ncopy)
        def _out():
            pltpu.sync_copy(
                den_acc.at[pl.ds(pl.multiple_of(s * rows_per_sub, 8), rows_per_sub)],
                den_o.at[c, pl.ds(pl.multiple_of(s * rows_per_sub, 8), rows_per_sub)])

    return k(src_i, dst_i, tka, tqa, ea16)


def _scb(src_i, dst_i, kf, qf, vpk, s2, s2v, p):
    e = src_i.shape[0]
    n = kf.shape[0]
    ew = e // NW
    cb = 40
    nchunks = ew // cb
    ncopy = 10
    rows_per_sub = n // ncopy
    f = jnp.float32
    mesh = plsc.VectorSubcoreMesh(core_axis_name="c", subcore_axis_name="s")

    @functools.partial(
        pl.kernel, mesh=mesh,
        out_type=[jax.ShapeDtypeStruct((e, HD), f),
                  jax.ShapeDtypeStruct((NC, n, HD), f)],
        scratch_types=[
            pltpu.VMEM((cb,), jnp.int32),
            pltpu.VMEM((cb,), jnp.int32),
            pltpu.VMEM((cb, HD), f),
            pltpu.VMEM((cb, HD), f),
            pltpu.VMEM((cb, HD), f),
            pltpu.VMEM((cb, HD), f),
            pltpu.VMEM((cb, HD), f),
            pltpu.VMEM((cb, 16), f),
            pltpu.VMEM_SHARED((n, HD), f),
            pltpu.SemaphoreType.DMA,
        ],
    )
    def k(src_h, dst_h, kf_h, qf_h, vpk_h, s2_h, s2v_h, p_h,
          oe_o, acc_o,
          idx_s, idx_d, kf_b, qf_b, vpk_b, s2_b, s2v_b, p_b,
          acc, sem):
        c = lax.axis_index("c")
        s = lax.axis_index("s")
        wid = s * NC + c

        @pl.when(s < ncopy)
        def _init():
            def zrow(j, carry):
                for t in range(H):
                    vpk_b[j, pl.ds(t * 16, 16)] = jnp.zeros((16,), f)
                return carry
            lax.fori_loop(0, cb, zrow, 0)
            for t in range(rows_per_sub // cb):
                pltpu.sync_copy(
                    vpk_b, acc.at[pl.ds(pl.multiple_of(s * rows_per_sub + t * cb, 8), cb)])
        plsc.subcore_barrier()

        def chunk(kk, carry):
            base = pl.multiple_of(wid * ew + kk * cb, 8)
            pltpu.sync_copy(src_h.at[pl.ds(base, cb)], idx_s)
            pltpu.sync_copy(dst_h.at[pl.ds(base, cb)], idx_d)
            d1 = pltpu.async_copy(kf_h.at[idx_s], kf_b, sem)
            d2 = pltpu.async_copy(qf_h.at[idx_d], qf_b, sem)
            d3 = pltpu.async_copy(vpk_h.at[idx_s], vpk_b, sem)
            d4 = pltpu.async_copy(s2_h.at[pl.ds(base, cb)], s2_b, sem)
            d5 = pltpu.async_copy(s2v_h.at[pl.ds(base, cb)], s2v_b, sem)
            d6 = pltpu.async_copy(p_h.at[pl.ds(base, cb)], p_b, sem)
            d1.wait()
            d2.wait()
            d3.wait()
            d4.wait()
            d5.wait()
            d6.wait()

            def ebody(j, cc):
                pv = p_b[j]
                for h in range(H):
                    sl = pl.ds(h * D, 16)
                    kf_b[j, sl] = kf_b[j, sl] + qf_b[j, sl] + s2_b[j, sl]
                    w = jnp.full((16,), pv[h], f)
                    vpk_b[j, sl] = w * (vpk_b[j, sl] + s2v_b[j, sl])
                return cc
            lax.fori_loop(0, cb, ebody, 0)

            pltpu.sync_copy(kf_b, oe_o.at[pl.ds(base, cb)])
            pltpu.sync_copy(vpk_b, acc.at[idx_d], add=True)
            return carry
        lax.fori_loop(0, nchunks, chunk, 0)
        plsc.subcore_barrier()

        @pl.when(s < ncopy)
        def _out():
            pltpu.sync_copy(
                acc.at[pl.ds(pl.multiple_of(s * rows_per_sub, 8), rows_per_sub)],
                acc_o.at[c, pl.ds(pl.multiple_of(s * rows_per_sub, 8), rows_per_sub)])

    return k(src_i, dst_i, kf, qf, vpk, s2, s2v, p)


def _tc3_body(a0, a1, d0, d1, qv2, r, out):
    den16 = d0[:, :16] + d1[:, :16]
    d128 = jnp.dot(den16, r[...], preferred_element_type=jnp.float32)
    inv = 1.0 / (d128 + 1e-16)
    out[...] = (a0[...] + a1[...]) * inv + (d128 * inv) * qv2[...]


def _tc3(acc0, acc1, den0, den1, qv2, r, nb):
    n = acc0.shape[0]
    blk = n // nb
    full = lambda s: pl.BlockSpec(s, lambda i: (0,) * len(s))
    row = pl.BlockSpec((blk, HD), lambda i: (i, 0))
    row16 = pl.BlockSpec((blk, 16), lambda i: (i, 0))
    return pl.pallas_call(
        _tc3_body,
        grid=(nb,),
        in_specs=[row, row, row, row, row, full((16, HD))],
        out_specs=row,
        out_shape=jax.ShapeDtypeStruct((n, HD), jnp.float32),
    )(acc0, acc1, den0, den1, qv2, r)


def kernel(x, edge_attr, WQ, bQ, WK, bK, WE, bE, WV, bV, Aw, VeRow, edge_index):
    n = x.shape[0]
    e = edge_attr.shape[0]
    f = jnp.float32

    wqt, wkt, wvt = WQ.T, WK.T, WV.T
    we_r = WE.reshape(H, 2, D, HD)
    we1t = we_r[:, 0].reshape(HD, HD).T
    we2t = we_r[:, 1].reshape(HD, HD).T
    be_r = bE.reshape(H, 2, D)
    be1 = be_r[:, 0].reshape(1, HD)
    be2 = be_r[:, 1].reshape(1, HD)
    eye8 = jnp.eye(H, dtype=f)
    m = jnp.einsum('dhc,hg->hdgc', VeRow, eye8).reshape(HD, HD)
    awm = jnp.einsum('dh,hg->hdg', Aw[:, :, 0], eye8).reshape(HD, H)
    awm = jnp.concatenate([awm, jnp.zeros((HD, H), f)], axis=1)
    awm128 = jnp.concatenate([awm, jnp.zeros((HD, HD - 16), f)], axis=1)
    r = jnp.concatenate(
        [jnp.kron(eye8, jnp.ones((1, D), f)), jnp.zeros((H, HD), f)], axis=0)
    bq = bQ.reshape(1, HD)
    bk = bK.reshape(1, HD)
    bv = bV.reshape(1, HD)
    src_i = edge_index[0]
    dst_i = edge_index[1]

    kf, qf, vpk, qv2, tka, tqa = _tc1(x, wqt, wkt, wvt, bq, bk, bv, m,
                                      awm128, 5)
    s2, s2v, ea16 = _tc2(edge_attr, we1t, we2t, be1, be2, m, awm, 160)
    p, denp = _sca(src_i, dst_i, tka, tqa, ea16)
    oe, accp = _scb(src_i, dst_i, kf, qf, vpk, s2, s2v, p)
    ov = _tc3(accp[0], accp[1], denp[0], denp[1], qv2, r, 5)
    return (ov, oe)


# speedup vs baseline: 44.5887x; 1.1634x over previous
import functools

import jax
import jax.numpy as jnp
from jax import lax
from jax.experimental import pallas as pl
from jax.experimental.pallas import tpu as pltpu
from jax.experimental.pallas import tpu_sc as plsc

H = 8
D = 16
HD = H * D
CLAMP = 5.0

NC = 2
NS = 16
NW = NC * NS


def _tc1_body(x_ref, wqt, wkt, wvt, bq, bk, bv, m, awm128, kf_o, qf_o, vpk_o,
              qv2_o, tka_o, tqa_o):
    xb = x_ref[...]
    kf = jnp.dot(xb, wkt[...], preferred_element_type=jnp.float32) + bk[...]
    qf = jnp.dot(xb, wqt[...], preferred_element_type=jnp.float32) + bq[...]
    vf = jnp.dot(xb, wvt[...], preferred_element_type=jnp.float32) + bv[...]
    kf_o[...] = kf
    qf_o[...] = qf
    vpk_o[...] = vf + jnp.dot(kf, m[...], preferred_element_type=jnp.float32)
    qv2_o[...] = jnp.dot(qf, m[...], preferred_element_type=jnp.float32)
    tka_o[...] = jnp.dot(kf, awm128[...], preferred_element_type=jnp.float32)
    tqa_o[...] = jnp.dot(qf, awm128[...], preferred_element_type=jnp.float32)


def _tc1(x, wqt, wkt, wvt, bq, bk, bv, m, awm128, nb):
    n = x.shape[0]
    blk = n // nb
    full = lambda s: pl.BlockSpec(s, lambda i: (0,) * len(s))
    row = pl.BlockSpec((blk, HD), lambda i: (i, 0))
    f = jnp.float32
    return pl.pallas_call(
        _tc1_body,
        grid=(nb,),
        in_specs=[row, full((HD, HD)), full((HD, HD)), full((HD, HD)),
                  full((1, HD)), full((1, HD)), full((1, HD)),
                  full((HD, HD)), full((HD, HD))],
        out_specs=[row, row, row, row, row, row],
        out_shape=[jax.ShapeDtypeStruct((n, HD), f)] * 6,
    )(x, wqt, wkt, wvt, bq, bk, bv, m, awm128)


def _tc2_body(ea_ref, we1t, we2t, be1, be2, m, awm, s2_o, s2v_o, eaa_o):
    eb = ea_ref[...]
    ex1 = jnp.dot(eb, we1t[...], preferred_element_type=jnp.float32) + be1[...]
    ex2 = jnp.dot(eb, we2t[...], preferred_element_type=jnp.float32) + be2[...]
    p2 = ex1 * ex2
    s2 = jnp.sign(p2) * jnp.sqrt(jnp.abs(p2))
    s2_o[...] = s2
    s2v_o[...] = jnp.dot(s2, m[...], preferred_element_type=jnp.float32)
    eaa_o[...] = jnp.dot(s2, awm[...], preferred_element_type=jnp.float32)


def _tc2(eattr, we1t, we2t, be1, be2, m, awm, nb):
    e = eattr.shape[0]
    blk = e // nb
    full = lambda s: pl.BlockSpec(s, lambda i: (0,) * len(s))
    row = pl.BlockSpec((blk, HD), lambda i: (i, 0))
    row16 = pl.BlockSpec((blk, 16), lambda i: (i, 0))
    f = jnp.float32
    return pl.pallas_call(
        _tc2_body,
        grid=(nb,),
        in_specs=[row, full((HD, HD)), full((HD, HD)), full((1, HD)),
                  full((1, HD)), full((HD, HD)), full((HD, 16))],
        out_specs=[row, row, row16],
        out_shape=[jax.ShapeDtypeStruct((e, HD), f),
                   jax.ShapeDtypeStruct((e, HD), f),
                   jax.ShapeDtypeStruct((e, 16), f)],
    )(eattr, we1t, we2t, be1, be2, m, awm)


def _sca(src_i, dst_i, tka, tqa, ea16):
    e = src_i.shape[0]
    n = tka.shape[0]
    ew = e // NW
    ca = 40
    nchunks = ew // ca
    ncopy = 10
    rows_per_sub = n // ncopy
    f = jnp.float32
    mesh = plsc.VectorSubcoreMesh(core_axis_name="c", subcore_axis_name="s")

    @functools.partial(
        pl.kernel, mesh=mesh,
        out_type=[jax.ShapeDtypeStruct((e, 16), f),
                  jax.ShapeDtypeStruct((NC, n, HD), f)],
        scratch_types=[
            pltpu.VMEM((ca,), jnp.int32),
            pltpu.VMEM((ca,), jnp.int32),
            pltpu.VMEM((ca, HD), f),
            pltpu.VMEM((ca, HD), f),
            pltpu.VMEM((ca, 16), f),
            pltpu.VMEM((ca,), jnp.int32),
            pltpu.VMEM((ca,), jnp.int32),
            pltpu.VMEM((ca, HD), f),
            pltpu.VMEM((ca, HD), f),
            pltpu.VMEM((ca, 16), f),
            pltpu.VMEM((ca, HD), f),
            pltpu.VMEM((ca, 16), f),
            pltpu.VMEM_SHARED((n, HD), f),
            pltpu.SemaphoreType.DMA,
            pltpu.SemaphoreType.DMA,
        ],
    )
    def k(src_h, dst_h, tka_h, tqa_h, ea_h, p_o, den_o,
          idx_s0, idx_d0, tka_b0, tqa_b0, ea_b0,
          idx_s1, idx_d1, tka_b1, tqa_b1, ea_b1,
          p_b, p16_b, den_acc, semA, semB):
        c = lax.axis_index("c")
        s = lax.axis_index("s")
        wid = s * NC + c
        mask = jnp.where(lax.iota(jnp.int32, 16) < 8, 1.0, 0.0).astype(f)
        sets = ((idx_s0, idx_d0, tka_b0, tqa_b0, ea_b0, semA),
                (idx_s1, idx_d1, tka_b1, tqa_b1, ea_b1, semB))

        def zrow(j, carry):
            for t in range(H):
                p_b[j, pl.ds(t * 16, 16)] = jnp.zeros((16,), f)
            return carry
        lax.fori_loop(0, ca, zrow, 0)

        @pl.when(s < ncopy)
        def _init():
            for t in range(rows_per_sub // ca):
                pltpu.sync_copy(
                    p_b, den_acc.at[pl.ds(pl.multiple_of(s * rows_per_sub + t * ca, 8), ca)])
        plsc.subcore_barrier()

        def fire(kk, st):
            idx_s, idx_d, tka_b, tqa_b, ea_b, sem = sets[st]
            base = pl.multiple_of(wid * ew + kk * ca, 8)
            pltpu.sync_copy(src_h.at[pl.ds(base, ca)], idx_s)
            pltpu.sync_copy(dst_h.at[pl.ds(base, ca)], idx_d)
            pltpu.async_copy(tka_h.at[idx_s], tka_b, sem)
            pltpu.async_copy(tqa_h.at[idx_d], tqa_b, sem)
            pltpu.async_copy(ea_h.at[pl.ds(base, ca)], ea_b, sem)

        def consume(kk, st):
            idx_s, idx_d, tka_b, tqa_b, ea_b, sem = sets[st]
            base = pl.multiple_of(wid * ew + kk * ca, 8)
            pltpu.make_async_copy(tka_h.at[idx_s], tka_b, sem).wait()
            pltpu.make_async_copy(tqa_h.at[idx_d], tqa_b, sem).wait()
            pltpu.make_async_copy(ea_h.at[pl.ds(base, ca)], ea_b, sem).wait()

            def body(j, cc):
                v = (tka_b[j, pl.ds(0, 16)] + tqa_b[j, pl.ds(0, 16)]
                     + ea_b[j])
                v = jnp.minimum(jnp.maximum(v, -CLAMP), CLAMP)
                pv = jnp.exp(v - CLAMP) * mask
                p_b[j, pl.ds(0, 16)] = pv
                p16_b[j] = pv
                return cc
            lax.fori_loop(0, ca, body, 0)
            pltpu.sync_copy(p16_b, p_o.at[pl.ds(base, ca)])
            pltpu.sync_copy(p_b, den_acc.at[idx_d], add=True)

        fire(0, 0)

        def pair(g, carry):
            k0 = 2 * g
            fire(k0 + 1, 1)
            consume(k0, 0)

            @pl.when(k0 + 2 < nchunks)
            def _pf():
                fire(k0 + 2, 0)
            consume(k0 + 1, 1)
            return carry
        lax.fori_loop(0, nchunks // 2, pair, 0)
        plsc.subcore_barrier()

        @pl.when(s < ncopy)
        def _out():
            pltpu.sync_copy(
                den_acc.at[pl.ds(pl.multiple_of(s * rows_per_sub, 8), rows_per_sub)],
                den_o.at[c, pl.ds(pl.multiple_of(s * rows_per_sub, 8), rows_per_sub)])

    return k(src_i, dst_i, tka, tqa, ea16)


def _scb(src_i, dst_i, kf, qf, vpk, s2, s2v, p):
    e = src_i.shape[0]
    n = kf.shape[0]
    ew = e // NW
    cb = 40
    nchunks = ew // cb
    ncopy = 10
    rows_per_sub = n // ncopy
    f = jnp.float32
    mesh = plsc.VectorSubcoreMesh(core_axis_name="c", subcore_axis_name="s")

    @functools.partial(
        pl.kernel, mesh=mesh,
        out_type=[jax.ShapeDtypeStruct((e, HD), f),
                  jax.ShapeDtypeStruct((NC, n, HD), f)],
        scratch_types=[
            pltpu.VMEM((cb,), jnp.int32),
            pltpu.VMEM((cb,), jnp.int32),
            pltpu.VMEM((cb, HD), f),
            pltpu.VMEM((cb, HD), f),
            pltpu.VMEM((cb, HD), f),
            pltpu.VMEM((cb, HD), f),
            pltpu.VMEM((cb, HD), f),
            pltpu.VMEM((cb, 16), f),
            pltpu.VMEM_SHARED((n, HD), f),
            pltpu.SemaphoreType.DMA,
        ],
    )
    def k(src_h, dst_h, kf_h, qf_h, vpk_h, s2_h, s2v_h, p_h,
          oe_o, acc_o,
          idx_s, idx_d, kf_b, qf_b, vpk_b, s2_b, s2v_b, p_b,
          acc, sem):
        c = lax.axis_index("c")
        s = lax.axis_index("s")
        wid = s * NC + c

        @pl.when(s < ncopy)
        def _init():
            def zrow(j, carry):
                for t in range(H):
                    vpk_b[j, pl.ds(t * 16, 16)] = jnp.zeros((16,), f)
                return carry
            lax.fori_loop(0, cb, zrow, 0)
            for t in range(rows_per_sub // cb):
                pltpu.sync_copy(
                    vpk_b, acc.at[pl.ds(pl.multiple_of(s * rows_per_sub + t * cb, 8), cb)])
        plsc.subcore_barrier()

        def chunk(kk, carry):
            base = pl.multiple_of(wid * ew + kk * cb, 8)
            pltpu.sync_copy(src_h.at[pl.ds(base, cb)], idx_s)
            pltpu.sync_copy(dst_h.at[pl.ds(base, cb)], idx_d)
            d1 = pltpu.async_copy(kf_h.at[idx_s], kf_b, sem)
            d2 = pltpu.async_copy(qf_h.at[idx_d], qf_b, sem)
            d3 = pltpu.async_copy(vpk_h.at[idx_s], vpk_b, sem)
            d4 = pltpu.async_copy(s2_h.at[pl.ds(base, cb)], s2_b, sem)
            d5 = pltpu.async_copy(s2v_h.at[pl.ds(base, cb)], s2v_b, sem)
            d6 = pltpu.async_copy(p_h.at[pl.ds(base, cb)], p_b, sem)
            d1.wait()
            d2.wait()
            d3.wait()
            d4.wait()
            d5.wait()
            d6.wait()

            def ebody(j, cc):
                pv = p_b[j]
                for h in range(H):
                    sl = pl.ds(h * D, 16)
                    kf_b[j, sl] = kf_b[j, sl] + qf_b[j, sl] + s2_b[j, sl]
                    w = jnp.full((16,), pv[h], f)
                    vpk_b[j, sl] = w * (vpk_b[j, sl] + s2v_b[j, sl])
                return cc
            lax.fori_loop(0, cb, ebody, 0)

            pltpu.sync_copy(kf_b, oe_o.at[pl.ds(base, cb)])
            pltpu.sync_copy(vpk_b, acc.at[idx_d], add=True)
            return carry
        lax.fori_loop(0, nchunks, chunk, 0)
        plsc.subcore_barrier()

        @pl.when(s < ncopy)
        def _out():
            pltpu.sync_copy(
                acc.at[pl.ds(pl.multiple_of(s * rows_per_sub, 8), rows_per_sub)],
                acc_o.at[c, pl.ds(pl.multiple_of(s * rows_per_sub, 8), rows_per_sub)])

    return k(src_i, dst_i, kf, qf, vpk, s2, s2v, p)


def _tc3_body(a0, a1, d0, d1, qv2, r, out):
    den16 = d0[:, :16] + d1[:, :16]
    d128 = jnp.dot(den16, r[...], preferred_element_type=jnp.float32)
    inv = 1.0 / (d128 + 1e-16)
    out[...] = (a0[...] + a1[...]) * inv + (d128 * inv) * qv2[...]


def _tc3(acc0, acc1, den0, den1, qv2, r, nb):
    n = acc0.shape[0]
    blk = n // nb
    full = lambda s: pl.BlockSpec(s, lambda i: (0,) * len(s))
    row = pl.BlockSpec((blk, HD), lambda i: (i, 0))
    row16 = pl.BlockSpec((blk, 16), lambda i: (i, 0))
    return pl.pallas_call(
        _tc3_body,
        grid=(nb,),
        in_specs=[row, row, row, row, row, full((16, HD))],
        out_specs=row,
        out_shape=jax.ShapeDtypeStruct((n, HD), jnp.float32),
    )(acc0, acc1, den0, den1, qv2, r)


def kernel(x, edge_attr, WQ, bQ, WK, bK, WE, bE, WV, bV, Aw, VeRow, edge_index):
    n = x.shape[0]
    e = edge_attr.shape[0]
    f = jnp.float32

    wqt, wkt, wvt = WQ.T, WK.T, WV.T
    we_r = WE.reshape(H, 2, D, HD)
    we1t = we_r[:, 0].reshape(HD, HD).T
    we2t = we_r[:, 1].reshape(HD, HD).T
    be_r = bE.reshape(H, 2, D)
    be1 = be_r[:, 0].reshape(1, HD)
    be2 = be_r[:, 1].reshape(1, HD)
    eye8 = jnp.eye(H, dtype=f)
    m = jnp.einsum('dhc,hg->hdgc', VeRow, eye8).reshape(HD, HD)
    awm = jnp.einsum('dh,hg->hdg', Aw[:, :, 0], eye8).reshape(HD, H)
    awm = jnp.concatenate([awm, jnp.zeros((HD, H), f)], axis=1)
    awm128 = jnp.concatenate([awm, jnp.zeros((HD, HD - 16), f)], axis=1)
    r = jnp.concatenate(
        [jnp.kron(eye8, jnp.ones((1, D), f)), jnp.zeros((H, HD), f)], axis=0)
    bq = bQ.reshape(1, HD)
    bk = bK.reshape(1, HD)
    bv = bV.reshape(1, HD)
    src_i = edge_index[0]
    dst_i = edge_index[1]

    kf, qf, vpk, qv2, tka, tqa = _tc1(x, wqt, wkt, wvt, bq, bk, bv, m,
                                      awm128, 5)
    s2, s2v, ea16 = _tc2(edge_attr, we1t, we2t, be1, be2, m, awm, 160)
    p, denp = _sca(src_i, dst_i, tka, tqa, ea16)
    oe, accp = _scb(src_i, dst_i, kf, qf, vpk, s2, s2v, p)
    ov = _tc3(accp[0], accp[1], denp[0], denp[1], qv2, r, 5)
    return (ov, oe)
